# Initial kernel scaffold; baseline (speedup 1.0000x reference)
import jax
import jax.numpy as jnp
from jax.experimental import pallas as pl


def kernel(x, edge_index, params):
    raise NotImplementedError("write your pallas kernel here")


# speedup vs baseline: 31.8687x; 31.8687x over previous
import functools

import jax
import jax.numpy as jnp
from jax import lax
from jax.experimental import pallas as pl
from jax.experimental.pallas import tpu as pltpu
from jax.experimental.pallas import tpu_sc as plsc

N_NODES = 10000
N_EDGES = 640000
T = 20
F = 16
H = 64
HEADS = 4
OUT = 16
G4 = 4 * H

BN = 400
BE = 2560
NB_N = N_NODES // BN

SC_NC = 2
SC_NS = 16
SC_NW = SC_NC * SC_NS
SC_K = 80

_f32 = jnp.float32


def _sig(x):
    return jax.nn.sigmoid(x)


def _lstm_kernel(xt_all, fw_row, w1, w2, projWf, projWb, projb):
    (w1f_ih, w1f_hh, b1f, w1b_ih, w1b_hh, b1b) = w1
    (w2f_ih_a, w2f_ih_b, w2f_hh, b2f, w2b_ih_a, w2b_ih_b, b2b) = w2

    def body(x_ref, fw_ref, w1fi_ref, w1fh_ref, b1f_ref, w1bi_ref, w1bh_ref,
             b1b_ref, w2fa_ref, w2fb_ref, w2fh_ref, b2f_ref, w2ba_ref,
             w2bb_ref, b2b_ref, pjf_ref, pjb_ref, pjb2_ref, out_ref,
             seqf, seqb, hs, cs):
        fw = fw_ref[...]

        def cell(gates, c):
            i = _sig(gates[:, 0:H])
            f = _sig(gates[:, H:2 * H])
            g = jnp.tanh(gates[:, 2 * H:3 * H])
            o = _sig(gates[:, 3 * H:4 * H])
            c2 = f * c + i * g
            h2 = o * jnp.tanh(c2)
            return h2, c2

        hs[...] = jnp.zeros((BN, H), _f32)
        cs[...] = jnp.zeros((BN, H), _f32)

        def l1f(t, _):
            xt = x_ref[t, :, :] * fw
            gates = (jnp.dot(xt, w1fi_ref[...].T, preferred_element_type=_f32)
                     + jnp.dot(hs[...], w1fh_ref[...].T,
                               preferred_element_type=_f32) + b1f_ref[...])
            h2, c2 = cell(gates, cs[...])
            hs[...] = h2
            cs[...] = c2
            seqf[t, :, :] = h2
            return 0

        lax.fori_loop(0, T, l1f, 0)

        hs[...] = jnp.zeros((BN, H), _f32)
        cs[...] = jnp.zeros((BN, H), _f32)

        def l1b(t, _):
            tr = T - 1 - t
            xt = x_ref[tr, :, :] * fw
            gates = (jnp.dot(xt, w1bi_ref[...].T, preferred_element_type=_f32)
                     + jnp.dot(hs[...], w1bh_ref[...].T,
                               preferred_element_type=_f32) + b1b_ref[...])
            h2, c2 = cell(gates, cs[...])
            hs[...] = h2
            cs[...] = c2
            seqb[tr, :, :] = h2
            return 0

        lax.fori_loop(0, T, l1b, 0)

        hs[...] = jnp.zeros((BN, H), _f32)
        cs[...] = jnp.zeros((BN, H), _f32)

        def l2f(t, _):
            gates = (jnp.dot(seqf[t, :, :], w2fa_ref[...].T, preferred_element_type=_f32)
                     + jnp.dot(seqb[t, :, :], w2fb_ref[...].T, preferred_element_type=_f32)
                     + jnp.dot(hs[...], w2fh_ref[...].T, preferred_element_type=_f32)
                     + b2f_ref[...])
            h2, c2 = cell(gates, cs[...])
            hs[...] = h2
            cs[...] = c2
            return 0

        lax.fori_loop(0, T, l2f, 0)
        hf2 = hs[...]

        gates = (jnp.dot(seqf[T - 1, :, :], w2ba_ref[...].T, preferred_element_type=_f32)
                 + jnp.dot(seqb[T - 1, :, :], w2bb_ref[...].T, preferred_element_type=_f32)
                 + b2b_ref[...])
        hb2, _ = cell(gates, jnp.zeros((BN, H), _f32))

        out_ref[...] = (jnp.dot(hf2, pjf_ref[...].T, preferred_element_type=_f32)
                        + jnp.dot(hb2, pjb_ref[...].T, preferred_element_type=_f32)
                        + pjb2_ref[...])

    full = lambda shape: pl.BlockSpec(shape, lambda i: (0,) * len(shape))
    return pl.pallas_call(
        body,
        grid=(NB_N,),
        in_specs=[
            pl.BlockSpec((T, BN, F), lambda i: (0, i, 0)),
            full((1, F)),
            full(w1f_ih.shape), full(w1f_hh.shape), full((1, G4)),
            full(w1b_ih.shape), full(w1b_hh.shape), full((1, G4)),
            full(w2f_ih_a.shape), full(w2f_ih_b.shape), full(w2f_hh.shape),
            full((1, G4)),
            full(w2b_ih_a.shape), full(w2b_ih_b.shape), full((1, G4)),
            full(projWf.shape), full(projWb.shape), full((1, H)),
        ],
        out_specs=pl.BlockSpec((BN, H), lambda i: (i, 0)),
        out_shape=jax.ShapeDtypeStruct((N_NODES, H), _f32),
        scratch_shapes=[
            pltpu.VMEM((T, BN, H), _f32),
            pltpu.VMEM((T, BN, H), _f32),
            pltpu.VMEM((BN, H), _f32),
            pltpu.VMEM((BN, H), _f32),
        ],
    )(xt_all, fw_row, w1f_ih, w1f_hh, b1f, w1b_ih, w1b_hh, b1b,
      w2f_ih_a, w2f_ih_b, w2f_hh, b2f, w2b_ih_a, w2b_ih_b, b2b,
      projWf, projWb, projb)


def _node_dense_kernel(h, W1a, W1b, b1, Wl, bl, Wr, br):
    def body(h_ref, w1a_ref, w1b_ref, b1_ref, wl_ref, bl_ref, wr_ref, br_ref,
             ts_ref, td_ref):
        hb = h_ref[...]
        ts_ref[:, 0:H] = jnp.dot(hb, w1a_ref[...].T, preferred_element_type=_f32)
        ts_ref[:, H:2 * H] = jnp.dot(hb, wl_ref[...].T,
                                     preferred_element_type=_f32) + bl_ref[...]
        td_ref[:, 0:H] = jnp.dot(hb, w1b_ref[...].T,
                                 preferred_element_type=_f32) + b1_ref[...]
        td_ref[:, H:2 * H] = jnp.dot(hb, wr_ref[...].T,
                                     preferred_element_type=_f32) + br_ref[...]

    full = lambda shape: pl.BlockSpec(shape, lambda i: (0,) * len(shape))
    ospec = pl.BlockSpec((BN, 2 * H), lambda i: (i, 0))
    oshape = jax.ShapeDtypeStruct((N_NODES, 2 * H), _f32)
    return pl.pallas_call(
        body,
        grid=(NB_N,),
        in_specs=[pl.BlockSpec((BN, H), lambda i: (i, 0)),
                  full((H, H)), full((H, H)), full((1, H)),
                  full((H, H)), full((1, H)), full((H, H)), full((1, H))],
        out_specs=[ospec, ospec],
        out_shape=[oshape, oshape],
    )(h, W1a, W1b, b1, Wl, bl, Wr, br)


def _edge_mlp_kernel(esrc, edst, W2, b2, W3p, b3p):
    nb = N_EDGES // BE

    def body(egs_ref, egd_ref, w2_ref, b2_ref, w3_ref, b3_ref, out_ref):
        z1 = jnp.tanh(egs_ref[:, 0:H] + egd_ref[:, 0:H])
        z2 = jnp.tanh(jnp.dot(z1, w2_ref[...].T,
                              preferred_element_type=_f32) + b2_ref[...])
        t16 = jnp.tanh(jnp.dot(z2, w3_ref[...],
                               preferred_element_type=_f32) + b3_ref[...])
        lane = lax.broadcasted_iota(jnp.int32, (BE, 16), 1)
        out_ref[...] = jnp.where(lane == 0, t16,
                                 jnp.where(lane == 1, 1.0, 0.0))

    full = lambda shape: pl.BlockSpec(shape, lambda i: (0,) * len(shape))
    return pl.pallas_call(
        body,
        grid=(nb,),
        in_specs=[pl.BlockSpec((BE, 2 * H), lambda i: (i, 0)),
                  pl.BlockSpec((BE, 2 * H), lambda i: (i, 0)),
                  full((H // 2, H)), full((1, H // 2)),
                  full((H // 2, 16)), full((1, 16))],
        out_specs=pl.BlockSpec((BE, 16), lambda i: (i, 0)),
        out_shape=jax.ShapeDtypeStruct((N_EDGES, 16), _f32),
    )(esrc, edst, W2, b2, W3p, b3p)


def _edge_all_kernel(esrc, edst, W2, b2, W3p, b3p,
                     We_row, att_row, S, ST):
    nb = N_EDGES // BE

    def body(es_ref, ed_ref, w2_ref, b2_ref, w3_ref,
             b3_ref, we_ref, att_ref, s_ref, st_ref,
             se_ref, ex_ref, nu_ref):
        lane = lax.broadcasted_iota(jnp.int32, (BE, 16), 1)
        es = es_ref[...]
        ed = ed_ref[...]
        z1 = jnp.tanh(es[:, 0:H] + ed[:, 0:H])
        z2 = jnp.tanh(jnp.dot(z1, w2_ref[...].T,
                              preferred_element_type=_f32) + b2_ref[...])
        t16 = jnp.tanh(jnp.dot(z2, w3_ref[...],
                               preferred_element_type=_f32) + b3_ref[...])
        se16 = jnp.where(lane == 0, t16, jnp.where(lane == 1, 1.0, 0.0))
        se_ref[...] = se16

        exl = es[:, H:2 * H]
        pre = exl + ed[:, H:2 * H] + se16[:, 0:1] * we_ref[...]
        m = jnp.where(pre > 0, pre, 0.2 * pre)
        a4 = jnp.dot(m * att_ref[...], s_ref[...], preferred_element_type=_f32)
        ex4 = jnp.exp(jnp.minimum(a4, 60.0))
        ex16 = jnp.concatenate([ex4, jnp.zeros((BE, 12), _f32)], axis=1)
        ex_ref[...] = ex16
        nu_ref[...] = exl * jnp.dot(ex4, st_ref[...],
                                    preferred_element_type=_f32)

    full = lambda shape: pl.BlockSpec(shape, lambda i: (0,) * len(shape))
    espec = pl.BlockSpec((BE, H), lambda i: (i, 0))
    e16 = pl.BlockSpec((BE, 16), lambda i: (i, 0))
    return pl.pallas_call(
        body,
        grid=(nb,),
        in_specs=[pl.BlockSpec((BE, 2 * H), lambda i: (i, 0)),
                  pl.BlockSpec((BE, 2 * H), lambda i: (i, 0)),
                  full((H // 2, H)), full((1, H // 2)),
                  full((H // 2, 16)), full((1, 16)),
                  full((1, H)), full((1, H)), full((H, 4)), full((4, H))],
        out_specs=[e16, e16, espec],
        out_shape=[jax.ShapeDtypeStruct((N_EDGES, 16), _f32),
                   jax.ShapeDtypeStruct((N_EDGES, 16), _f32),
                   jax.ShapeDtypeStruct((N_EDGES, H), _f32)],
    )(esrc, edst, W2, b2, W3p, b3p, We_row, att_row, S, ST)


def _node_final_kernel(h, xl, xr, s0, s1, d0, d1, c0, c1,
                       We_row, att_row, S, ST, bias_row, g_row, b_row):
    def body(h_ref, xl_ref, xr_ref, s0_ref, s1_ref, d0_ref, d1_ref,
             c0_ref, c1_ref, we_ref, att_ref, s_ref, st_ref,
             bias_ref, g_ref, b_ref, out_ref):
        xl = xl_ref[...]
        sums = s0_ref[...] + s1_ref[...]
        la = sums[:, 0:1] / jnp.maximum(sums[:, 1:2], 1.0)
        pre = xl + xr_ref[...] + la * we_ref[...]
        m = jnp.where(pre > 0, pre, 0.2 * pre)
        a4 = jnp.dot(m * att_ref[...], s_ref[...], preferred_element_type=_f32)
        exl4 = jnp.exp(jnp.minimum(a4, 60.0))
        denom4 = d0_ref[:, 0:HEADS] + d1_ref[:, 0:HEADS] + exl4 + 1e-16
        numer = (c0_ref[...] + c1_ref[...]
                 + xl * jnp.dot(exl4, st_ref[...], preferred_element_type=_f32))
        out64 = numer / jnp.dot(denom4, st_ref[...],
                                preferred_element_type=_f32)
        y = h_ref[...] + out64 + bias_ref[...]
        mu = jnp.mean(y, axis=1, keepdims=True)
        d = y - mu
        var = jnp.mean(d * d, axis=1, keepdims=True)
        out_ref[...] = d * lax.rsqrt(var + 1e-5) * g_ref[...] + b_ref[...]

    full = lambda shape: pl.BlockSpec(shape, lambda i: (0,) * len(shape))
    nspec = pl.BlockSpec((BN, H), lambda i: (i, 0))
    n16 = pl.BlockSpec((BN, 16), lambda i: (i, 0))
    return pl.pallas_call(
        body,
        grid=(NB_N,),
        in_specs=[nspec, nspec, nspec, n16, n16, n16, n16, nspec, nspec,
                  full((1, H)), full((1, H)),
                  full((H, 4)), full((4, H)),
                  full((1, H)), full((1, H)), full((1, H))],
        out_specs=nspec,
        out_shape=jax.ShapeDtypeStruct((N_NODES, H), _f32),
    )(h, xl, xr, s0, s1, d0, d1, c0, c1,
      We_row, att_row, S, ST, bias_row, g_row, b_row)


def _head_kernel(j1, j2, jkWa, jkWb, jk_b, jkg, jkb2, p1W, p1b, p1g, p1b2,
                 p2W, p2b, p2g, p2b2, p3W, p3b):
    inv = float((1.0 + 1e-5) ** -0.5)

    def body(j1_ref, j2_ref, jka_ref, jkb_ref, jkbias_ref, jkg_ref, jkb2_ref,
             p1w_ref, p1b_ref, p1g_ref, p1b2_ref, p2w_ref, p2b_ref, p2g_ref,
             p2b2_ref, p3w_ref, p3b_ref, hf_ref, pred_ref):
        t = (jnp.dot(j1_ref[...], jka_ref[...].T, preferred_element_type=_f32)
             + jnp.dot(j2_ref[...], jkb_ref[...].T, preferred_element_type=_f32)
             + jkbias_ref[...])
        hf = jnp.maximum(t * inv * jkg_ref[...] + jkb2_ref[...], 0.0)
        hf_ref[...] = hf
        z = jnp.dot(hf, p1w_ref[...].T, preferred_element_type=_f32) + p1b_ref[...]
        z = jnp.maximum(z * inv * p1g_ref[...] + p1b2_ref[...], 0.0)
        z = jnp.dot(z, p2w_ref[...].T, preferred_element_type=_f32) + p2b_ref[...]
        z = jnp.maximum(z * inv * p2g_ref[...] + p2b2_ref[...], 0.0)
        pred_ref[...] = jnp.dot(z, p3w_ref[...],
                                preferred_element_type=_f32) + p3b_ref[...]

    full = lambda shape: pl.BlockSpec(shape, lambda i: (0,) * len(shape))
    nspec = pl.BlockSpec((BN, H), lambda i: (i, 0))
    return pl.pallas_call(
        body,
        grid=(NB_N,),
        in_specs=[nspec, nspec,
                  full((H, H)), full((H, H)), full((1, H)), full((1, H)),
                  full((1, H)),
                  full((H, H)), full((1, H)), full((1, H)), full((1, H)),
                  full((H // 2, H)), full((1, H // 2)), full((1, H // 2)),
                  full((1, H // 2)),
                  full((H // 2, 16)), full((1, 16))],
        out_specs=[nspec, pl.BlockSpec((BN, 16), lambda i: (i, 0))],
        out_shape=[jax.ShapeDtypeStruct((N_NODES, H), _f32),
                   jax.ShapeDtypeStruct((N_NODES, 16), _f32)],
    )(j1, j2, jkWa, jkWb, jk_b, jkg, jkb2, p1W, p1b, p1g, p1b2,
      p2W, p2b, p2g, p2b2, p3W, p3b)


@functools.lru_cache(maxsize=None)
def _sc_gather_multi_fn(idx_sel, D):
    NT = len(idx_sel)
    EW = N_EDGES // SC_NW
    NCH = EW // SC_K
    need = sorted(set(idx_sel))
    mesh = plsc.VectorSubcoreMesh(core_axis_name="c", subcore_axis_name="s")

    scratch = [pltpu.VMEM((2, SC_K), jnp.int32) for _ in range(2)]
    scratch += [pltpu.VMEM((2, SC_K, D), _f32) for _ in range(NT)]
    scratch += [pltpu.SemaphoreType.DMA for _ in range(4)]

    @functools.partial(
        pl.kernel,
        out_type=[jax.ShapeDtypeStruct((N_EDGES, D), _f32)] * NT,
        mesh=mesh,
        compiler_params=pltpu.CompilerParams(use_tc_tiling_on_sc=False,
                                             skip_device_barrier=True),
        scratch_types=scratch,
    )
    def k(*refs):
        tabs = refs[0:NT]
        idxs = refs[NT:NT + 2]
        outs = refs[NT + 2:2 * NT + 2]
        sc = refs[2 * NT + 2:]
        iv = sc[0:2]
        rows = sc[2:2 + NT]
        gsem = sc[2 + NT:4 + NT]
        wsem = sc[4 + NT:6 + NT]

        wid = lax.axis_index("s") * SC_NC + lax.axis_index("c")
        base = wid * EW

        def load_idx(j, b):
            for u in need:
                pltpu.sync_copy(idxs[u].at[pl.ds(base + j * SC_K, SC_K)],
                                iv[u].at[b])

        def start_gathers(b):
            for t in range(NT):
                pltpu.async_copy(tabs[t].at[iv[idx_sel[t]].at[b]],
                                 rows[t].at[b], gsem[b])

        def drain_gathers(b):
            for t in range(NT):
                pltpu.make_async_copy(tabs[t].at[iv[idx_sel[t]].at[b]],
                                      rows[t].at[b], gsem[b]).wait()

        def start_writes(j, b):
            for t in range(NT):
                pltpu.async_copy(rows[t].at[b],
                                 outs[t].at[pl.ds(base + j * SC_K, SC_K)],
                                 wsem[b])

        def drain_writes(j, b):
            for t in range(NT):
                pltpu.make_async_copy(rows[t].at[b],
                                      outs[t].at[pl.ds(base + j * SC_K, SC_K)],
                                      wsem[b]).wait()

        for b in range(2):
            load_idx(b, b)
            start_gathers(b)

        @pl.loop(0, NCH - 2, step=2)
        def _(i):
            for b in range(2):
                j = i + b
                drain_gathers(b)
                start_writes(j, b)
                load_idx(j + 2, b)
                drain_writes(j, b)
                start_gathers(b)

        for b in range(2):
            j = NCH - 2 + b
            drain_gathers(b)
            for t in range(NT):
                pltpu.sync_copy(rows[t].at[b],
                                outs[t].at[pl.ds(base + j * SC_K, SC_K)])

    return k


def _sc_gather2(ta, tb, src, dst):
    return _sc_gather_multi_fn((0, 1), ta.shape[1])(ta, tb, src, dst)


@functools.lru_cache(maxsize=None)
def _sc_scatter_multi_fn(widths):
    NT = len(widths)
    EW = N_EDGES // SC_NW
    NCH = EW // SC_K
    NZ = N_NODES // SC_K
    mesh = plsc.VectorSubcoreMesh(core_axis_name="c", subcore_axis_name="s")

    scratch = [pltpu.VMEM_SHARED((N_NODES, D), _f32) for D in widths]
    scratch += [pltpu.VMEM((SC_K, D), _f32) for D in widths]
    scratch += [pltpu.VMEM((2, SC_K), jnp.int32)]
    scratch += [pltpu.VMEM((2, SC_K, D), _f32) for D in widths]
    scratch += [pltpu.SemaphoreType.DMA for _ in range(4)]

    @functools.partial(
        pl.kernel,
        out_type=[jax.ShapeDtypeStruct((2 * N_NODES, D), _f32)
                  for D in widths],
        mesh=mesh,
        compiler_params=pltpu.CompilerParams(use_tc_tiling_on_sc=False,
                                             skip_device_barrier=True),
        scratch_types=scratch,
    )
    def k(*refs):
        vals_hbm = refs[0:NT]
        idx_hbm = refs[NT]
        outs = refs[NT + 1:2 * NT + 1]
        sc = refs[2 * NT + 1:]
        accs = sc[0:NT]
        zbs = sc[NT:2 * NT]
        idx_v = sc[2 * NT]
        vals_v = sc[2 * NT + 1:3 * NT + 1]
        lsem = sc[3 * NT + 1:3 * NT + 3]
        ssem = sc[3 * NT + 3:3 * NT + 5]

        c = lax.axis_index("c")
        s = lax.axis_index("s")
        wid = s * SC_NC + c

        @pl.loop(0, SC_K)
        def _(r):
            for t in range(NT):
                for j in range(widths[t] // 16):
                    zbs[t][r, pl.ds(j * 16, 16)] = jnp.zeros((16,), _f32)

        @pl.loop(0, NZ)
        def _(j):
            @pl.when(lax.rem(j, SC_NS) == s)
            def _():
                for t in range(NT):
                    pltpu.sync_copy(zbs[t], accs[t].at[pl.ds(j * SC_K, SC_K)])

        plsc.subcore_barrier()

        def start_loads(j, b):
            off = wid * EW + j * SC_K
            pltpu.async_copy(idx_hbm.at[pl.ds(off, SC_K)], idx_v.at[b],
                             lsem[b])
            for t in range(NT):
                pltpu.async_copy(vals_hbm[t].at[pl.ds(off, SC_K)],
                                 vals_v[t].at[b], lsem[b])

        def drain_loads(j, b):
            off = wid * EW + j * SC_K
            pltpu.make_async_copy(idx_hbm.at[pl.ds(off, SC_K)], idx_v.at[b],
                                  lsem[b]).wait()
            for t in range(NT):
                pltpu.make_async_copy(vals_hbm[t].at[pl.ds(off, SC_K)],
                                      vals_v[t].at[b], lsem[b]).wait()

        def start_scatters(b):
            for t in range(NT):
                pltpu.async_copy(vals_v[t].at[b], accs[t].at[idx_v.at[b]],
                                 ssem[b], add=True)

        def drain_scatters(b):
            for t in range(NT):
                pltpu.make_async_copy(vals_v[t].at[b], accs[t].at[idx_v.at[b]],
                                      ssem[b]).wait()

        for b in range(2):
            start_loads(b, b)

        @pl.loop(0, NCH - 2, step=2)
        def _(i):
            for b in range(2):
                j = i + b
                drain_loads(j, b)
                start_scatters(b)
                drain_scatters(b)
                start_loads(j + 2, b)

        for b in range(2):
            j = NCH - 2 + b
            drain_loads(j, b)
            start_scatters(b)
            drain_scatters(b)

        plsc.subcore_barrier()

        @pl.loop(0, NZ)
        def _(j):
            @pl.when(lax.rem(j, SC_NS) == s)
            def _():
                for t in range(NT):
                    pltpu.sync_copy(
                        accs[t].at[pl.ds(j * SC_K, SC_K)],
                        outs[t].at[pl.ds(c * N_NODES + j * SC_K, SC_K)])

    return k


def _sc_scatter3(v16a, v16b, v64, idx):
    return _sc_scatter_multi_fn((16, 16, 64))(v16a, v16b, v64, idx)


def kernel(x, edge_index, params):
    src = edge_index[0]
    dst = edge_index[1]
    p = params

    fw_row = p['feature_weights'].reshape(1, F)
    xt_all = jnp.swapaxes(x, 0, 1)

    l1, l2 = p['lstm']
    w1 = (l1['fwd']['Wih'], l1['fwd']['Whh'],
          (l1['fwd']['bih'] + l1['fwd']['bhh']).reshape(1, G4),
          l1['bwd']['Wih'], l1['bwd']['Whh'],
          (l1['bwd']['bih'] + l1['bwd']['bhh']).reshape(1, G4))
    w2 = (l2['fwd']['Wih'][:, :H], l2['fwd']['Wih'][:, H:], l2['fwd']['Whh'],
          (l2['fwd']['bih'] + l2['fwd']['bhh']).reshape(1, G4),
          l2['bwd']['Wih'][:, :H], l2['bwd']['Wih'][:, H:],
          (l2['bwd']['bih'] + l2['bwd']['bhh']).reshape(1, G4))

    h = _lstm_kernel(xt_all, fw_row, w1, w2,
                     p['proj_W'][:, :H], p['proj_W'][:, H:],
                     p['proj_b'].reshape(1, H))

    ep = p['edge']
    W1a = ep['W1'][:, :H]
    W1b = ep['W1'][:, H:]
    b1 = ep['b1'].reshape(1, H)
    W2 = ep['W2']
    b2 = ep['b2'].reshape(1, H // 2)
    W3p = jnp.zeros((H // 2, 16), _f32).at[:, 0].set(ep['W3'][0])
    b3p = jnp.zeros((1, 16), _f32).at[0, 0].set(ep['b3'][0])

    col = jnp.arange(H, dtype=jnp.int32) // OUT
    row = jnp.arange(HEADS, dtype=jnp.int32)
    S = (col[:, None] == row[None, :]).astype(_f32)
    ST = S.T

    jk = []
    se16 = None
    for l in range(2):
        g = p['gat'][l]
        tsrc, tdst = _node_dense_kernel(
            h, W1a, W1b, b1, g['Wl'], g['bl'].reshape(1, H),
            g['Wr'], g['br'].reshape(1, H))
        esrc, edst = _sc_gather2(tsrc, tdst, src, dst)
        We_row = g['We'].reshape(1, H)
        att_row = g['att'].reshape(1, H)
        se16, ex16, numer = _edge_all_kernel(
            esrc, edst, W2, b2, W3p, b3p, We_row, att_row, S, ST)
        sp, dp, cp = _sc_scatter3(se16, ex16, numer, dst)

        xl = tsrc[:, H:]
        xr = tdst[:, H:]
        h = _node_final_kernel(
            h, xl, xr, sp[:N_NODES], sp[N_NODES:], dp[:N_NODES], dp[N_NODES:],
            cp[:N_NODES], cp[N_NODES:], We_row, att_row, S, ST,
            g['bias'].reshape(1, H),
            p['ln'][l]['g'].reshape(1, H),
            p['ln'][l]['b'].reshape(1, H))
        jk.append(h)

    tsrc, tdst = _node_dense_kernel(
        h, W1a, W1b, b1, p['gat'][0]['Wl'], p['gat'][0]['bl'].reshape(1, H),
        p['gat'][0]['Wr'], p['gat'][0]['br'].reshape(1, H))
    esrc, edst = _sc_gather2(tsrc, tdst, src, dst)
    se16f = _edge_mlp_kernel(esrc, edst, W2, b2, W3p, b3p)
    se = se16f[:, 0]

    hf, pred = _head_kernel(
        jk[0], jk[1],
        p['jk_W'][:, :H], p['jk_W'][:, H:], p['jk_b'].reshape(1, H),
        p['jk_bn_g'].reshape(1, H), p['jk_bn_b'].reshape(1, H),
        p['p1_W'], p['p1_b'].reshape(1, H), p['p1_bn_g'].reshape(1, H),
        p['p1_bn_b'].reshape(1, H),
        p['p2_W'], p['p2_b'].reshape(1, H // 2),
        p['p2_bn_g'].reshape(1, H // 2), p['p2_bn_b'].reshape(1, H // 2),
        jnp.zeros((H // 2, 16), _f32).at[:, 0].set(p['p3_W'][0]),
        jnp.zeros((1, 16), _f32).at[0, 0].set(p['p3_b'][0]))

    return pred[:, 0], hf, p['feature_weights'], se
